# baseline (speedup 1.0000x reference)
import functools

import jax
import jax.numpy as jnp
from jax import lax
from jax.experimental import pallas as pl
from jax.experimental.pallas import tpu as pltpu
from jax.experimental.pallas import tpu_sc as plsc

_SIZE = 100000
_CATS = 100
_DIM = 64
_E = 1024

_BV = 2048


@functools.partial(jax.jit, static_argnums=(2, 3))
def _sc_gather(table, idx, B, D):
    info = plsc.get_sparse_core_info()
    NW = info.num_cores * info.num_subcores
    b_per_w = B // NW
    mesh = plsc.VectorSubcoreMesh(core_axis_name="c", subcore_axis_name="s")

    @functools.partial(
        pl.kernel,
        mesh=mesh,
        out_type=jax.ShapeDtypeStruct((B, D), jnp.float32),
        scratch_types=[
            pltpu.VMEM((b_per_w,), jnp.int32),
            pltpu.VMEM((b_per_w, D), jnp.float32),
            pltpu.SemaphoreType.DMA,
        ],
        compiler_params=pltpu.CompilerParams(use_tc_tiling_on_sc=False),
    )
    def k(table_hbm, idx_hbm, out_hbm, idx_v, rows_v, sem):
        wid = lax.axis_index("s") * info.num_cores + lax.axis_index("c")
        base = wid * b_per_w
        pltpu.sync_copy(idx_hbm.at[pl.ds(base, b_per_w)], idx_v)
        pltpu.async_copy(table_hbm.at[idx_v], rows_v, sem).wait()
        pltpu.sync_copy(rows_v, out_hbm.at[pl.ds(base, b_per_w)])

    return k(table, idx)


def _head_body(phiw_ref, phic_ref, wct_ref, wc_ref, bc_ref, g_ref,
               prior_ref, z_ref, nd_ref):
    phiw = phiw_ref[...]
    wct = wct_ref[...]
    bc = bc_ref[...]
    cw = phiw * phic_ref[...]
    logits = jnp.dot(cw, wct, preferred_element_type=jnp.float32) + bc
    y = jax.nn.softmax(logits + g_ref[...], axis=-1)
    cols = lax.broadcasted_iota(jnp.int32, (_E, _CATS), 1)
    ymax = jnp.max(y, axis=-1, keepdims=True)
    cand = jnp.where(y >= ymax, cols, jnp.int32(2**30))
    first = jnp.min(cand, axis=-1, keepdims=True)
    z = (cols == first).astype(jnp.float32)
    z_ref[...] = z
    pl_logits = jnp.dot(phiw, wct, preferred_element_type=jnp.float32) + bc
    prior_ref[...] = jax.nn.softmax(pl_logits, axis=-1)
    nd_ref[...] = jnp.dot(z, wc_ref[...], preferred_element_type=jnp.float32)


def _dec_body(nd_ref, wd_ref, bd_ref, out_ref):
    out_ref[...] = lax.dot_general(
        nd_ref[...], wd_ref[...], (((1,), (1,)), ((), ())),
        preferred_element_type=jnp.float32) + bd_ref[...]


def kernel(w, c, edge_index, node_emb, W_comm, b_comm, W_dec, b_dec):
    del edge_index
    idx_all = jnp.concatenate([w, c]).astype(jnp.int32)
    phi = _sc_gather(node_emb, idx_all, 2 * _E, _DIM)
    phi_w, phi_c = phi[:_E], phi[_E:]

    g = jax.random.gumbel(jax.random.key(42), (_E, _CATS), jnp.float32)
    bc2 = b_comm.reshape(1, _CATS)
    prior, z, nd = pl.pallas_call(
        _head_body,
        out_shape=(
            jax.ShapeDtypeStruct((_E, _CATS), jnp.float32),
            jax.ShapeDtypeStruct((_E, _CATS), jnp.float32),
            jax.ShapeDtypeStruct((_E, _DIM), jnp.float32),
        ),
    )(phi_w, phi_c, W_comm.T, W_comm, bc2, g)

    bd2 = b_dec.reshape(1, _SIZE)
    nb = pl.cdiv(_SIZE, _BV)
    recon_c = pl.pallas_call(
        _dec_body,
        grid=(nb,),
        in_specs=[
            pl.BlockSpec((_E, _DIM), lambda i: (0, 0)),
            pl.BlockSpec((_BV, _DIM), lambda i: (i, 0)),
            pl.BlockSpec((1, _BV), lambda i: (0, i)),
        ],
        out_specs=pl.BlockSpec((_E, _BV), lambda i: (0, i)),
        out_shape=jax.ShapeDtypeStruct((_E, _SIZE), jnp.float32),
    )(nd, W_dec, bd2)

    return (prior, recon_c, z)


# speedup vs baseline: 1.0062x; 1.0062x over previous
import functools

import jax
import jax.numpy as jnp
from jax import lax
from jax.experimental import pallas as pl
from jax.experimental.pallas import tpu as pltpu
from jax.experimental.pallas import tpu_sc as plsc

_SIZE = 100000
_CATS = 100
_DIM = 64
_E = 1024

_BV = 4096


@functools.partial(jax.jit, static_argnums=(2, 3))
def _sc_gather(table, idx, B, D):
    info = plsc.get_sparse_core_info()
    NW = info.num_cores * info.num_subcores
    b_per_w = B // NW
    mesh = plsc.VectorSubcoreMesh(core_axis_name="c", subcore_axis_name="s")

    @functools.partial(
        pl.kernel,
        mesh=mesh,
        out_type=jax.ShapeDtypeStruct((B, D), jnp.float32),
        scratch_types=[
            pltpu.VMEM((b_per_w,), jnp.int32),
            pltpu.VMEM((b_per_w, D), jnp.float32),
            pltpu.SemaphoreType.DMA,
        ],
        compiler_params=pltpu.CompilerParams(use_tc_tiling_on_sc=False),
    )
    def k(table_hbm, idx_hbm, out_hbm, idx_v, rows_v, sem):
        wid = lax.axis_index("s") * info.num_cores + lax.axis_index("c")
        base = wid * b_per_w
        pltpu.sync_copy(idx_hbm.at[pl.ds(base, b_per_w)], idx_v)
        pltpu.async_copy(table_hbm.at[idx_v], rows_v, sem).wait()
        pltpu.sync_copy(rows_v, out_hbm.at[pl.ds(base, b_per_w)])

    return k(table, idx)


def _head_body(phiw_ref, phic_ref, wct_ref, wc_ref, bc_ref, g_ref,
               prior_ref, z_ref, nd_ref):
    phiw = phiw_ref[...]
    wct = wct_ref[...]
    bc = bc_ref[...]
    cw = phiw * phic_ref[...]
    logits = jnp.dot(cw, wct, preferred_element_type=jnp.float32) + bc
    y = jax.nn.softmax(logits + g_ref[...], axis=-1)
    cols = lax.broadcasted_iota(jnp.int32, (_E, _CATS), 1)
    ymax = jnp.max(y, axis=-1, keepdims=True)
    cand = jnp.where(y >= ymax, cols, jnp.int32(2**30))
    first = jnp.min(cand, axis=-1, keepdims=True)
    z = (cols == first).astype(jnp.float32)
    z_ref[...] = z
    pl_logits = jnp.dot(phiw, wct, preferred_element_type=jnp.float32) + bc
    prior_ref[...] = jax.nn.softmax(pl_logits, axis=-1)
    nd_ref[...] = jnp.dot(z, wc_ref[...], preferred_element_type=jnp.float32)


def _dec_body(nd_ref, wd_ref, bd_ref, out_ref):
    out_ref[...] = lax.dot_general(
        nd_ref[...], wd_ref[...], (((1,), (1,)), ((), ())),
        preferred_element_type=jnp.float32) + bd_ref[...]


def kernel(w, c, edge_index, node_emb, W_comm, b_comm, W_dec, b_dec):
    del edge_index
    idx_all = jnp.concatenate([w, c]).astype(jnp.int32)
    phi = _sc_gather(node_emb, idx_all, 2 * _E, _DIM)
    phi_w, phi_c = phi[:_E], phi[_E:]

    g = jax.random.gumbel(jax.random.key(42), (_E, _CATS), jnp.float32)
    bc2 = b_comm.reshape(1, _CATS)
    prior, z, nd = pl.pallas_call(
        _head_body,
        out_shape=(
            jax.ShapeDtypeStruct((_E, _CATS), jnp.float32),
            jax.ShapeDtypeStruct((_E, _CATS), jnp.float32),
            jax.ShapeDtypeStruct((_E, _DIM), jnp.float32),
        ),
    )(phi_w, phi_c, W_comm.T, W_comm, bc2, g)

    bd2 = b_dec.reshape(1, _SIZE)
    nb = pl.cdiv(_SIZE, _BV)
    recon_c = pl.pallas_call(
        _dec_body,
        grid=(nb,),
        in_specs=[
            pl.BlockSpec((_E, _DIM), lambda i: (0, 0)),
            pl.BlockSpec((_BV, _DIM), lambda i: (i, 0)),
            pl.BlockSpec((1, _BV), lambda i: (0, i)),
        ],
        out_specs=pl.BlockSpec((_E, _BV), lambda i: (0, i)),
        out_shape=jax.ShapeDtypeStruct((_E, _SIZE), jnp.float32),
        compiler_params=pltpu.CompilerParams(
            dimension_semantics=("parallel",)),
    )(nd, W_dec, bd2)

    return (prior, recon_c, z)


# speedup vs baseline: 1.8996x; 1.8880x over previous
import functools

import jax
import jax.numpy as jnp
from jax import lax
from jax.experimental import pallas as pl
from jax.experimental.pallas import tpu as pltpu
from jax.experimental.pallas import tpu_sc as plsc

_SIZE = 100000
_CATS = 100
_DIM = 64
_E = 1024

_BV = 4096


@functools.partial(jax.jit, static_argnums=(2, 3))
def _sc_gather(table, idx, B, D):
    info = plsc.get_sparse_core_info()
    NW = info.num_cores * info.num_subcores
    b_per_w = B // NW
    mesh = plsc.VectorSubcoreMesh(core_axis_name="c", subcore_axis_name="s")

    @functools.partial(
        pl.kernel,
        mesh=mesh,
        out_type=jax.ShapeDtypeStruct((B, D), jnp.float32),
        scratch_types=[
            pltpu.VMEM((b_per_w,), jnp.int32),
            pltpu.VMEM((b_per_w, D), jnp.float32),
            pltpu.SemaphoreType.DMA,
        ],
        compiler_params=pltpu.CompilerParams(use_tc_tiling_on_sc=False),
    )
    def k(table_hbm, idx_hbm, out_hbm, idx_v, rows_v, sem):
        wid = lax.axis_index("s") * info.num_cores + lax.axis_index("c")
        base = wid * b_per_w
        pltpu.sync_copy(idx_hbm.at[pl.ds(base, b_per_w)], idx_v)
        pltpu.async_copy(table_hbm.at[idx_v], rows_v, sem).wait()
        pltpu.sync_copy(rows_v, out_hbm.at[pl.ds(base, b_per_w)])

    return k(table, idx)


def _head_body(phiw_ref, phic_ref, wct_ref, wc_ref, bc_ref, g_ref,
               prior_ref, z_ref, nd_ref):
    phiw = phiw_ref[...]
    wct = wct_ref[...]
    bc = bc_ref[...]
    cw = phiw * phic_ref[...]
    logits = jnp.dot(cw, wct, preferred_element_type=jnp.float32) + bc
    y = jax.nn.softmax(logits + g_ref[...], axis=-1)
    cols = lax.broadcasted_iota(jnp.int32, (_E, _CATS), 1)
    ymax = jnp.max(y, axis=-1, keepdims=True)
    cand = jnp.where(y >= ymax, cols, jnp.int32(2**30))
    first = jnp.min(cand, axis=-1, keepdims=True)
    z = (cols == first).astype(jnp.float32)
    z_ref[...] = z
    pl_logits = jnp.dot(phiw, wct, preferred_element_type=jnp.float32) + bc
    prior_ref[...] = jax.nn.softmax(pl_logits, axis=-1)
    nd_ref[...] = jnp.dot(z, wc_ref[...], preferred_element_type=jnp.float32)


def _dec_body(wd_ref, nd_ref, bd_ref, out_ref):
    out_ref[...] = lax.dot_general(
        wd_ref[...], nd_ref[...], (((1,), (1,)), ((), ())),
        preferred_element_type=jnp.float32) + bd_ref[...]


def kernel(w, c, edge_index, node_emb, W_comm, b_comm, W_dec, b_dec):
    del edge_index
    idx_all = jnp.concatenate([w, c]).astype(jnp.int32)
    phi = _sc_gather(node_emb, idx_all, 2 * _E, _DIM)
    phi_w, phi_c = phi[:_E], phi[_E:]

    g = jax.random.gumbel(jax.random.key(42), (_E, _CATS), jnp.float32)
    bc2 = b_comm.reshape(1, _CATS)
    prior, z, nd = pl.pallas_call(
        _head_body,
        out_shape=(
            jax.ShapeDtypeStruct((_E, _CATS), jnp.float32),
            jax.ShapeDtypeStruct((_E, _CATS), jnp.float32),
            jax.ShapeDtypeStruct((_E, _DIM), jnp.float32),
        ),
    )(phi_w, phi_c, W_comm.T, W_comm, bc2, g)

    bd2 = b_dec.reshape(_SIZE, 1)
    nb = pl.cdiv(_SIZE, _BV)
    recon_t = pl.pallas_call(
        _dec_body,
        grid=(nb,),
        in_specs=[
            pl.BlockSpec((_BV, _DIM), lambda i: (i, 0)),
            pl.BlockSpec((_E, _DIM), lambda i: (0, 0)),
            pl.BlockSpec((_BV, 1), lambda i: (i, 0)),
        ],
        out_specs=pl.BlockSpec((_BV, _E), lambda i: (i, 0)),
        out_shape=jax.ShapeDtypeStruct((_SIZE, _E), jnp.float32),
        compiler_params=pltpu.CompilerParams(
            dimension_semantics=("parallel",)),
    )(W_dec, nd, bd2)
    recon_c = recon_t.T

    return (prior, recon_c, z)


# speedup vs baseline: 2.2313x; 1.1746x over previous
import functools

import jax
import jax.numpy as jnp
from jax import lax
from jax.experimental import pallas as pl
from jax.experimental.pallas import tpu as pltpu
from jax.experimental.pallas import tpu_sc as plsc

_SIZE = 100000
_CATS = 100
_DIM = 64
_E = 1024

_BV = 4096


@functools.partial(jax.jit, static_argnums=(2, 3))
def _sc_gather(table, idx, B, D):
    info = plsc.get_sparse_core_info()
    NW = info.num_cores * info.num_subcores
    b_per_w = B // NW
    mesh = plsc.VectorSubcoreMesh(core_axis_name="c", subcore_axis_name="s")

    @functools.partial(
        pl.kernel,
        mesh=mesh,
        out_type=jax.ShapeDtypeStruct((B, D), jnp.float32),
        scratch_types=[
            pltpu.VMEM((b_per_w,), jnp.int32),
            pltpu.VMEM((b_per_w, D), jnp.float32),
            pltpu.SemaphoreType.DMA,
        ],
        compiler_params=pltpu.CompilerParams(use_tc_tiling_on_sc=False),
    )
    def k(table_hbm, idx_hbm, out_hbm, idx_v, rows_v, sem):
        wid = lax.axis_index("s") * info.num_cores + lax.axis_index("c")
        base = wid * b_per_w
        pltpu.sync_copy(idx_hbm.at[pl.ds(base, b_per_w)], idx_v)
        pltpu.async_copy(table_hbm.at[idx_v], rows_v, sem).wait()
        pltpu.sync_copy(rows_v, out_hbm.at[pl.ds(base, b_per_w)])

    return k(table, idx)


def _head_body(phiw_ref, phic_ref, wc_ref, bct_ref, gt_ref,
               priort_ref, zt_ref, nd_ref):
    phiw = phiw_ref[...]
    wc = wc_ref[...]
    bct = bct_ref[...]
    cw = phiw * phic_ref[...]
    logits_t = lax.dot_general(
        wc, cw, (((1,), (1,)), ((), ())),
        preferred_element_type=jnp.float32) + bct
    yt = jax.nn.softmax(logits_t + gt_ref[...], axis=0)
    rows = lax.broadcasted_iota(jnp.int32, (_CATS, _E), 0)
    ymax = jnp.max(yt, axis=0, keepdims=True)
    cand = jnp.where(yt >= ymax, rows, jnp.int32(2**30))
    first = jnp.min(cand, axis=0, keepdims=True)
    zt = (rows == first).astype(jnp.float32)
    zt_ref[...] = zt
    pl_t = lax.dot_general(
        wc, phiw, (((1,), (1,)), ((), ())),
        preferred_element_type=jnp.float32) + bct
    priort_ref[...] = jax.nn.softmax(pl_t, axis=0)
    nd_ref[...] = lax.dot_general(
        zt, wc, (((0,), (0,)), ((), ())),
        preferred_element_type=jnp.float32)


def _dec_body(wdt_ref, nd_ref, bd_ref, out_ref):
    out_ref[...] = lax.dot_general(
        wdt_ref[...], nd_ref[...], (((0,), (1,)), ((), ())),
        preferred_element_type=jnp.float32) + bd_ref[...]


def kernel(w, c, edge_index, node_emb, W_comm, b_comm, W_dec, b_dec):
    del edge_index
    idx_all = jnp.concatenate([w, c]).astype(jnp.int32)
    phi = _sc_gather(node_emb, idx_all, 2 * _E, _DIM)
    phi_w, phi_c = phi[:_E], phi[_E:]

    gt = jax.random.gumbel(jax.random.key(42), (_E, _CATS), jnp.float32).T
    bct = b_comm.reshape(_CATS, 1)
    prior_t, z_t, nd = pl.pallas_call(
        _head_body,
        out_shape=(
            jax.ShapeDtypeStruct((_CATS, _E), jnp.float32),
            jax.ShapeDtypeStruct((_CATS, _E), jnp.float32),
            jax.ShapeDtypeStruct((_E, _DIM), jnp.float32),
        ),
    )(phi_w, phi_c, W_comm, bct, gt)

    bd2 = b_dec.reshape(_SIZE, 1)
    nb = pl.cdiv(_SIZE, _BV)
    recon_t = pl.pallas_call(
        _dec_body,
        grid=(nb,),
        in_specs=[
            pl.BlockSpec((_DIM, _BV), lambda i: (0, i)),
            pl.BlockSpec((_E, _DIM), lambda i: (0, 0)),
            pl.BlockSpec((_BV, 1), lambda i: (i, 0)),
        ],
        out_specs=pl.BlockSpec((_BV, _E), lambda i: (i, 0)),
        out_shape=jax.ShapeDtypeStruct((_SIZE, _E), jnp.float32),
        compiler_params=pltpu.CompilerParams(
            dimension_semantics=("parallel",)),
    )(W_dec.T, nd, bd2)

    return (prior_t.T, recon_t.T, z_t.T)


# speedup vs baseline: 2.8119x; 1.2602x over previous
import functools

import jax
import jax.numpy as jnp
from jax import lax
from jax.experimental import pallas as pl
from jax.experimental.pallas import tpu as pltpu
from jax.experimental.pallas import tpu_sc as plsc

_SIZE = 100000
_CATS = 100
_DIM = 64
_E = 1024

_BV = 4096


@functools.partial(jax.jit, static_argnums=(2, 3))
def _sc_gather(table, idx, B, D):
    info = plsc.get_sparse_core_info()
    NW = info.num_cores * info.num_subcores
    b_per_w = B // NW
    mesh = plsc.VectorSubcoreMesh(core_axis_name="c", subcore_axis_name="s")

    @functools.partial(
        pl.kernel,
        mesh=mesh,
        out_type=jax.ShapeDtypeStruct((B, D), jnp.float32),
        scratch_types=[
            pltpu.VMEM((b_per_w,), jnp.int32),
            pltpu.VMEM((b_per_w, D), jnp.float32),
            pltpu.SemaphoreType.DMA,
        ],
        compiler_params=pltpu.CompilerParams(use_tc_tiling_on_sc=False),
    )
    def k(table_hbm, idx_hbm, out_hbm, idx_v, rows_v, sem):
        wid = lax.axis_index("s") * info.num_cores + lax.axis_index("c")
        base = wid * b_per_w
        pltpu.sync_copy(idx_hbm.at[pl.ds(base, b_per_w)], idx_v)
        pltpu.async_copy(table_hbm.at[idx_v], rows_v, sem).wait()
        pltpu.sync_copy(rows_v, out_hbm.at[pl.ds(base, b_per_w)])

    return k(table, idx)


def _head_body(phiw_ref, phic_ref, wc_ref, bct_ref, gt_ref,
               priort_ref, zt_ref, nd_ref):
    phiw = phiw_ref[...]
    wc = wc_ref[...]
    bct = bct_ref[...]
    cw = phiw * phic_ref[...]
    logits_t = lax.dot_general(
        wc, cw, (((1,), (1,)), ((), ())),
        preferred_element_type=jnp.float32) + bct
    yt = jax.nn.softmax(logits_t + gt_ref[...], axis=0)
    rows = lax.broadcasted_iota(jnp.int32, (_CATS, _E), 0)
    ymax = jnp.max(yt, axis=0, keepdims=True)
    cand = jnp.where(yt >= ymax, rows, jnp.int32(2**30))
    first = jnp.min(cand, axis=0, keepdims=True)
    zt = (rows == first).astype(jnp.float32)
    zt_ref[...] = zt
    pl_t = lax.dot_general(
        wc, phiw, (((1,), (1,)), ((), ())),
        preferred_element_type=jnp.float32) + bct
    priort_ref[...] = jax.nn.softmax(pl_t, axis=0)
    ndv = lax.dot_general(
        zt, wc, (((0,), (0,)), ((), ())),
        preferred_element_type=jnp.float32)
    nd_ref[...] = jnp.concatenate(
        [ndv, jnp.ones((_E, 1), jnp.float32)], axis=1)


def _dec_body(wdt_ref, bdr_ref, nd_ref, out_ref):
    lhs = jnp.concatenate([wdt_ref[...], bdr_ref[...]], axis=0)
    out_ref[...] = lax.dot_general(
        lhs, nd_ref[...], (((0,), (1,)), ((), ())),
        preferred_element_type=jnp.float32)


def kernel(w, c, edge_index, node_emb, W_comm, b_comm, W_dec, b_dec):
    del edge_index
    idx_all = jnp.concatenate([w, c]).astype(jnp.int32)
    phi = _sc_gather(node_emb, idx_all, 2 * _E, _DIM)
    phi_w, phi_c = phi[:_E], phi[_E:]

    gt = jax.random.gumbel(jax.random.key(42), (_E, _CATS), jnp.float32).T
    bct = b_comm.reshape(_CATS, 1)
    prior_t, z_t, nd = pl.pallas_call(
        _head_body,
        out_shape=(
            jax.ShapeDtypeStruct((_CATS, _E), jnp.float32),
            jax.ShapeDtypeStruct((_CATS, _E), jnp.float32),
            jax.ShapeDtypeStruct((_E, _DIM + 1), jnp.float32),
        ),
    )(phi_w, phi_c, W_comm, bct, gt)

    bdr = b_dec.reshape(1, _SIZE)
    nb = pl.cdiv(_SIZE, _BV)
    recon_t = pl.pallas_call(
        _dec_body,
        grid=(nb,),
        in_specs=[
            pl.BlockSpec((_DIM, _BV), lambda i: (0, i)),
            pl.BlockSpec((1, _BV), lambda i: (0, i)),
            pl.BlockSpec((_E, _DIM + 1), lambda i: (0, 0)),
        ],
        out_specs=pl.BlockSpec((_BV, _E), lambda i: (i, 0)),
        out_shape=jax.ShapeDtypeStruct((_SIZE, _E), jnp.float32),
        compiler_params=pltpu.CompilerParams(
            dimension_semantics=("parallel",)),
    )(W_dec.T, bdr, nd)

    return (prior_t.T, recon_t.T, z_t.T)


# speedup vs baseline: 2.8198x; 1.0028x over previous
import functools

import jax
import jax.numpy as jnp
from jax import lax
from jax.experimental import pallas as pl
from jax.experimental.pallas import tpu as pltpu
from jax.experimental.pallas import tpu_sc as plsc

_SIZE = 100000
_CATS = 100
_DIM = 64
_E = 1024

_BV = 5120


@functools.partial(jax.jit, static_argnums=(2, 3))
def _sc_gather(table, idx, B, D):
    info = plsc.get_sparse_core_info()
    NW = info.num_cores * info.num_subcores
    b_per_w = B // NW
    mesh = plsc.VectorSubcoreMesh(core_axis_name="c", subcore_axis_name="s")

    @functools.partial(
        pl.kernel,
        mesh=mesh,
        out_type=jax.ShapeDtypeStruct((B, D), jnp.float32),
        scratch_types=[
            pltpu.VMEM((b_per_w,), jnp.int32),
            pltpu.VMEM((b_per_w, D), jnp.float32),
            pltpu.SemaphoreType.DMA,
        ],
        compiler_params=pltpu.CompilerParams(use_tc_tiling_on_sc=False),
    )
    def k(table_hbm, idx_hbm, out_hbm, idx_v, rows_v, sem):
        wid = lax.axis_index("s") * info.num_cores + lax.axis_index("c")
        base = wid * b_per_w
        pltpu.sync_copy(idx_hbm.at[pl.ds(base, b_per_w)], idx_v)
        pltpu.async_copy(table_hbm.at[idx_v], rows_v, sem).wait()
        pltpu.sync_copy(rows_v, out_hbm.at[pl.ds(base, b_per_w)])

    return k(table, idx)


def _head_body(phiw_ref, phic_ref, wc_ref, bct_ref, gt_ref,
               priort_ref, zt_ref, nd_ref):
    phiw = phiw_ref[...]
    wc = wc_ref[...]
    bct = bct_ref[...]
    cw = phiw * phic_ref[...]
    logits_t = lax.dot_general(
        wc, cw, (((1,), (1,)), ((), ())),
        preferred_element_type=jnp.float32) + bct
    yt = jax.nn.softmax(logits_t + gt_ref[...], axis=0)
    rows = lax.broadcasted_iota(jnp.int32, (_CATS, _E), 0)
    ymax = jnp.max(yt, axis=0, keepdims=True)
    cand = jnp.where(yt >= ymax, rows, jnp.int32(2**30))
    first = jnp.min(cand, axis=0, keepdims=True)
    zt = (rows == first).astype(jnp.float32)
    zt_ref[...] = zt
    pl_t = lax.dot_general(
        wc, phiw, (((1,), (1,)), ((), ())),
        preferred_element_type=jnp.float32) + bct
    priort_ref[...] = jax.nn.softmax(pl_t, axis=0)
    ndv = lax.dot_general(
        zt, wc, (((0,), (0,)), ((), ())),
        preferred_element_type=jnp.float32)
    nd_ref[...] = jnp.concatenate(
        [ndv, jnp.ones((_E, 1), jnp.float32)], axis=1)


def _dec_body(wdt_ref, bdr_ref, nd_ref, out_ref):
    lhs = jnp.concatenate([wdt_ref[...], bdr_ref[...]], axis=0)
    out_ref[...] = lax.dot_general(
        lhs, nd_ref[...], (((0,), (1,)), ((), ())),
        preferred_element_type=jnp.float32)


def kernel(w, c, edge_index, node_emb, W_comm, b_comm, W_dec, b_dec):
    del edge_index
    idx_all = jnp.concatenate([w, c]).astype(jnp.int32)
    phi = _sc_gather(node_emb, idx_all, 2 * _E, _DIM)
    phi_w, phi_c = phi[:_E], phi[_E:]

    gt = jax.random.gumbel(jax.random.key(42), (_E, _CATS), jnp.float32).T
    bct = b_comm.reshape(_CATS, 1)
    prior_t, z_t, nd = pl.pallas_call(
        _head_body,
        out_shape=(
            jax.ShapeDtypeStruct((_CATS, _E), jnp.float32),
            jax.ShapeDtypeStruct((_CATS, _E), jnp.float32),
            jax.ShapeDtypeStruct((_E, _DIM + 1), jnp.float32),
        ),
    )(phi_w, phi_c, W_comm, bct, gt)

    bdr = b_dec.reshape(1, _SIZE)
    nb = pl.cdiv(_SIZE, _BV)
    recon_t = pl.pallas_call(
        _dec_body,
        grid=(nb,),
        in_specs=[
            pl.BlockSpec((_DIM, _BV), lambda i: (0, i)),
            pl.BlockSpec((1, _BV), lambda i: (0, i)),
            pl.BlockSpec((_E, _DIM + 1), lambda i: (0, 0)),
        ],
        out_specs=pl.BlockSpec((_BV, _E), lambda i: (i, 0)),
        out_shape=jax.ShapeDtypeStruct((_SIZE, _E), jnp.float32),
        compiler_params=pltpu.CompilerParams(
            dimension_semantics=("parallel",)),
    )(W_dec.T, bdr, nd)

    return (prior_t.T, recon_t.T, z_t.T)
